# Initial kernel scaffold; baseline (speedup 1.0000x reference)
import jax
import jax.numpy as jnp
from jax.experimental import pallas as pl


def kernel(edge_index, emb_users, emb_items):
    raise NotImplementedError("write your pallas kernel here")


# speedup vs baseline: 4.8987x; 4.8987x over previous
import functools

import jax
import jax.numpy as jnp
from jax import lax
from jax.experimental import pallas as pl
from jax.experimental.pallas import tpu as pltpu
from jax.experimental.pallas import tpu_sc as plsc

NUM_USERS = 25000
NUM_ITEMS = 25000
N = NUM_USERS + NUM_ITEMS
D = 64
E = 800000
NUM_LAYERS = 4

NC, NS, LANES = 2, 16, 16
HALF = 25088
N_PAD = 2 * HALF
TILE_ROWS = HALF // NS
SUB = 56
NSUB = TILE_ROWS // SUB

EPT = 51200
E_PAD = EPT * NS
IDX_W = 128
BLK = 4
NBLK = EPT // (BLK * IDX_W)
ROWS_PT = EPT // IDX_W

PAD_ROW = N_PAD - 8
PAD_COL = N_PAD + 64
DUMMY = HALF

_mesh = plsc.VectorSubcoreMesh(
    core_axis_name="c", subcore_axis_name="s", num_cores=NC, num_subcores=NS)
_sc_params = pltpu.CompilerParams(use_tc_tiling_on_sc=False)


def _vloop(n, body):
    lax.fori_loop(0, n, lambda i, _: (body(i), None)[1], None)


def _localize(col_ref, loc_ref, base):
    def step(i):
        jr = i // (IDX_W // LANES)
        jc = i % (IDX_W // LANES)
        sl = pl.ds(jc * LANES, LANES)
        c = col_ref[jr, sl] - base
        ok = (c >= 0) & (c < HALF)
        loc_ref[jr, sl] = jnp.where(ok, c, DUMMY)
    _vloop(BLK * (IDX_W // LANES), step)


def _fill2d(ref, rows, cols, value):
    def step(i):
        r = i // (cols // LANES)
        q = i % (cols // LANES)
        ref[r, pl.ds(q * LANES, LANES)] = jnp.full((LANES,), value, jnp.float32)
    _vloop(rows * (cols // LANES), step)


@functools.partial(
    pl.kernel,
    out_type=jax.ShapeDtypeStruct((N_PAD, 16), jnp.float32),
    mesh=_mesh,
    compiler_params=_sc_params,
    scratch_types=[
        pltpu.VMEM_SHARED((HALF + 8, 16), jnp.float32),
        pltpu.VMEM((BLK, IDX_W), jnp.int32),
        pltpu.VMEM((BLK, IDX_W), jnp.int32),
        pltpu.VMEM((IDX_W, 16), jnp.float32),
        pltpu.VMEM((SUB, 16), jnp.float32),
    ],
)
def _deg_kernel(col_hbm, deg_out, dS, colb, locb, ones, zb):
    core = lax.axis_index("c")
    s = lax.axis_index("s")
    base = core * HALF

    _fill2d(ones, IDX_W, 16, 1.0)
    _fill2d(zb, SUB, 16, 0.0)
    for cchunk in range(NSUB):
        pltpu.sync_copy(zb, dS.at[pl.ds(s * TILE_ROWS + cchunk * SUB, SUB)])
    plsc.subcore_barrier()

    def block(b):
        r0 = s * ROWS_PT + b * BLK
        pltpu.sync_copy(col_hbm.at[pl.ds(r0, BLK)], colb)
        _localize(colb, locb, base)
        for j in range(BLK):
            pltpu.sync_copy(ones, dS.at[locb.at[j]], add=True)
    _vloop(NBLK, block)

    plsc.subcore_barrier()
    lrow = s * TILE_ROWS
    pltpu.sync_copy(dS.at[pl.ds(lrow, TILE_ROWS)],
                    deg_out.at[pl.ds(base + lrow, TILE_ROWS)])


def _dis_body(deg_ref, emb_ref, dis_ref, y0_ref):
    d = deg_ref[:, 0:1]
    r = lax.rsqrt(d)
    disc = jnp.where(d > 0, r, 0.0)
    dis_ref[...] = jnp.broadcast_to(disc, dis_ref.shape)
    y0_ref[...] = emb_ref[...] * disc


_K1_ROWS = 256
_dis_call = pl.pallas_call(
    _dis_body,
    grid=(N_PAD // _K1_ROWS,),
    in_specs=[
        pl.BlockSpec((_K1_ROWS, 16), lambda i: (i, 0)),
        pl.BlockSpec((_K1_ROWS, D), lambda i: (i, 0)),
    ],
    out_specs=[pl.BlockSpec((_K1_ROWS, D), lambda i: (i, 0))] * 2,
    out_shape=[jax.ShapeDtypeStruct((N_PAD, D), jnp.float32)] * 2,
)


def _make_layer(scale):
    @functools.partial(
        pl.kernel,
        out_type=[jax.ShapeDtypeStruct((N_PAD, D), jnp.float32),
                  jax.ShapeDtypeStruct((N_PAD, D), jnp.float32)],
        mesh=_mesh,
        compiler_params=_sc_params,
        scratch_types=[
            pltpu.VMEM_SHARED((HALF + 8, D), jnp.float32),
            pltpu.VMEM((BLK, IDX_W), jnp.int32),
            pltpu.VMEM((BLK, IDX_W), jnp.int32),
            pltpu.VMEM((BLK, IDX_W), jnp.int32),
            pltpu.VMEM((2, IDX_W, D), jnp.float32),
            pltpu.VMEM((SUB, D), jnp.float32),
            pltpu.VMEM((SUB, D), jnp.float32),
            pltpu.VMEM((SUB, D), jnp.float32),
            pltpu.SemaphoreType.DMA,
        ],
    )
    def layer(row_hbm, col_hbm, y_hbm, dis_hbm, acc_hbm,
              y_out, acc_out,
              sS, rowb, colb, locb, gb, sbuf, dbuf, abuf, gsem):
        core = lax.axis_index("c")
        s = lax.axis_index("s")
        base = core * HALF

        _fill2d(sbuf, SUB, D, 0.0)
        for cchunk in range(NSUB):
            pltpu.sync_copy(sbuf, sS.at[pl.ds(s * TILE_ROWS + cchunk * SUB, SUB)])
        plsc.subcore_barrier()

        def block(b):
            r0 = s * ROWS_PT + b * BLK
            pltpu.sync_copy(row_hbm.at[pl.ds(r0, BLK)], rowb)
            pltpu.sync_copy(col_hbm.at[pl.ds(r0, BLK)], colb)
            _localize(colb, locb, base)
            d0 = pltpu.async_copy(y_hbm.at[rowb.at[0]], gb.at[0], gsem)
            d1 = pltpu.async_copy(y_hbm.at[rowb.at[1]], gb.at[1], gsem)
            d0.wait()
            pltpu.sync_copy(gb.at[0], sS.at[locb.at[0]], add=True)
            d2 = pltpu.async_copy(y_hbm.at[rowb.at[2]], gb.at[0], gsem)
            d1.wait()
            pltpu.sync_copy(gb.at[1], sS.at[locb.at[1]], add=True)
            d3 = pltpu.async_copy(y_hbm.at[rowb.at[3]], gb.at[1], gsem)
            d2.wait()
            pltpu.sync_copy(gb.at[0], sS.at[locb.at[2]], add=True)
            d3.wait()
            pltpu.sync_copy(gb.at[1], sS.at[locb.at[3]], add=True)
        _vloop(NBLK, block)
        plsc.subcore_barrier()

        def chunk(cchunk):
            lrow = s * TILE_ROWS + cchunk * SUB
            g = base + lrow
            pltpu.sync_copy(sS.at[pl.ds(lrow, SUB)], sbuf)
            pltpu.sync_copy(dis_hbm.at[pl.ds(g, SUB)], dbuf)
            pltpu.sync_copy(acc_hbm.at[pl.ds(g, SUB)], abuf)

            def step(i):
                r = i // (D // LANES)
                q = i % (D // LANES)
                sl = pl.ds(q * LANES, LANES)
                dv = dbuf[r, sl]
                x = sbuf[r, sl] * dv
                a = abuf[r, sl] + x
                if scale != 1.0:
                    a = a * scale
                abuf[r, sl] = a
                sbuf[r, sl] = x * dv
            _vloop(SUB * (D // LANES), step)

            pltpu.sync_copy(sbuf, y_out.at[pl.ds(g, SUB)])
            pltpu.sync_copy(abuf, acc_out.at[pl.ds(g, SUB)])
        _vloop(NSUB, chunk)

    return layer


_layer_mid = _make_layer(1.0)
_layer_last = _make_layer(1.0 / (NUM_LAYERS + 1) ** 2)


def kernel(edge_index, emb_users, emb_items):
    row = edge_index[0].astype(jnp.int32)
    col = edge_index[1].astype(jnp.int32)
    row_p = jnp.concatenate(
        [row, jnp.full((E_PAD - E,), PAD_ROW, jnp.int32)]).reshape(-1, IDX_W)
    col_p = jnp.concatenate(
        [col, jnp.full((E_PAD - E,), PAD_COL, jnp.int32)]).reshape(-1, IDX_W)
    emb = jnp.concatenate([emb_users, emb_items], axis=0)
    emb_pad = jnp.concatenate(
        [emb, jnp.zeros((N_PAD - N, D), jnp.float32)], axis=0)

    deg16 = _deg_kernel(col_p)
    dis, y0 = _dis_call(deg16, emb_pad)

    y, acc = y0, emb_pad
    for l in range(NUM_LAYERS):
        fn = _layer_last if l == NUM_LAYERS - 1 else _layer_mid
        y, acc = fn(row_p, col_p, y, dis, acc)

    return (acc[:NUM_USERS], emb_users, acc[NUM_USERS:N], emb_items)


# speedup vs baseline: 9.7376x; 1.9878x over previous
import functools

import jax
import jax.numpy as jnp
from jax import lax
from jax.experimental import pallas as pl
from jax.experimental.pallas import tpu as pltpu
from jax.experimental.pallas import tpu_sc as plsc

NUM_USERS = 25000
NUM_ITEMS = 25000
N = NUM_USERS + NUM_ITEMS
D = 64
HD = D // 2
E = 800000
NUM_LAYERS = 4

NC, NS, LANES = 2, 16, 16
N_PAD = 50176
STRIPE = N_PAD // NS

E_PAD = 819200
ROWS_ALL = E_PAD // 128
RPT = ROWS_ALL // NS
RB = RPT // 4
RPT0 = ROWS_ALL // (NC * NS)
RB0 = RPT0 // 4

PAD_ROW = N_PAD - 8
PAD_COL = N_PAD

_mesh = plsc.VectorSubcoreMesh(
    core_axis_name="c", subcore_axis_name="s", num_cores=NC, num_subcores=NS)
_sc_params = pltpu.CompilerParams(use_tc_tiling_on_sc=False)


def _vloop(n, body):
    lax.fori_loop(0, n, lambda i, _: (body(i), None)[1], None)


def _fill2d(ref, rows, cols, value):
    def step(i):
        r = i // (cols // LANES)
        q = i % (cols // LANES)
        ref[r, pl.ds(q * LANES, LANES)] = jnp.full((LANES,), value, jnp.float32)
    _vloop(rows * (cols // LANES), step)


def _zero_stripe(sS, zb, stripe_base):
    for k in range(STRIPE // 128):
        pltpu.sync_copy(zb, sS.at[pl.ds(stripe_base + k * 128, 128)])
    pltpu.sync_copy(zb.at[pl.ds(0, 64)],
                    sS.at[pl.ds(stripe_base + (STRIPE // 128) * 128, 64)])


@functools.partial(
    pl.kernel,
    out_type=jax.ShapeDtypeStruct((2 * N_PAD, 16), jnp.float32),
    mesh=_mesh,
    compiler_params=_sc_params,
    scratch_types=[
        pltpu.VMEM_SHARED((N_PAD + 8, 16), jnp.float32),
        pltpu.VMEM((4, 128), jnp.int32),
        pltpu.VMEM((4, 128), jnp.int32),
        pltpu.VMEM((128, 16), jnp.float32),
        pltpu.VMEM((128, 16), jnp.float32),
        pltpu.SemaphoreType.DMA,
        pltpu.SemaphoreType.DMA,
        pltpu.SemaphoreType.DMA,
    ],
)
def _deg_kernel(col_hbm, deg_out, dS, colb0, colb1, ones, zb,
                isem0, isem1, ssem):
    core = lax.axis_index("c")
    s = lax.axis_index("s")

    _fill2d(ones, 128, 16, 1.0)
    _fill2d(zb, 128, 16, 0.0)
    _zero_stripe(dS, zb, s * STRIPE)
    plsc.subcore_barrier()

    tbase = (core * NS + s) * RPT0
    colbs, isems = (colb0, colb1), (isem0, isem1)

    def idx_copy(b, p):
        return pltpu.make_async_copy(
            col_hbm.at[pl.ds(tbase + b * 4, 4)], colbs[p], isems[p])

    def round_(b, p, last):
        idx_copy(b, p).wait()
        if not last:
            idx_copy(b + 1, 1 - p).start()
        for j in range(4):
            pltpu.make_async_copy(
                ones, dS.at[colbs[p].at[j]], ssem).start(add=True)

    idx_copy(0, 0).start()
    round_(0, 0, False)

    def pair(r, _):
        round_(2 * r + 1, 1, False)
        round_(2 * r + 2, 0, False)
        return None
    lax.fori_loop(0, (RB0 - 2) // 2, pair, None)
    round_(RB0 - 1, 1, True)

    _vloop(RB0 * 4, lambda i: pltpu.make_async_copy(
        ones, dS.at[colb0.at[0]], ssem).wait())

    plsc.subcore_barrier()
    lrow = s * STRIPE
    pltpu.sync_copy(dS.at[pl.ds(lrow, STRIPE)],
                    deg_out.at[pl.ds(core * N_PAD + lrow, STRIPE)])


_K1_ROWS = 256


def _dis_body(deg_ref, emb_ref, dis_ref, y2_ref):
    d = deg_ref[0, :, 0:1] + deg_ref[1, :, 0:1]
    disc = jnp.where(d > 0, lax.rsqrt(d), 0.0)
    dis_ref[...] = jnp.broadcast_to(disc, dis_ref.shape)
    y2_ref[0] = emb_ref[:, :HD] * disc
    y2_ref[1] = emb_ref[:, HD:] * disc


_dis_call = pl.pallas_call(
    _dis_body,
    grid=(N_PAD // _K1_ROWS,),
    in_specs=[
        pl.BlockSpec((2, _K1_ROWS, 16), lambda i: (0, i, 0)),
        pl.BlockSpec((_K1_ROWS, D), lambda i: (i, 0)),
    ],
    out_specs=[
        pl.BlockSpec((_K1_ROWS, 16), lambda i: (i, 0)),
        pl.BlockSpec((2, _K1_ROWS, HD), lambda i: (0, i, 0)),
    ],
    out_shape=[
        jax.ShapeDtypeStruct((N_PAD, 16), jnp.float32),
        jax.ShapeDtypeStruct((2, N_PAD, HD), jnp.float32),
    ],
)


@functools.partial(
    pl.kernel,
    out_type=jax.ShapeDtypeStruct((2 * N_PAD, HD), jnp.float32),
    mesh=_mesh,
    compiler_params=_sc_params,
    scratch_types=[
        pltpu.VMEM_SHARED((N_PAD + 8, HD), jnp.float32),
        pltpu.VMEM((4, 128), jnp.int32),
        pltpu.VMEM((4, 128), jnp.int32),
        pltpu.VMEM((4, 128), jnp.int32),
        pltpu.VMEM((4, 128), jnp.int32),
        pltpu.VMEM((4, 128, HD), jnp.float32),
        pltpu.VMEM((128, HD), jnp.float32),
        pltpu.SemaphoreType.DMA,
        pltpu.SemaphoreType.DMA,
        pltpu.SemaphoreType.DMA,
        pltpu.SemaphoreType.DMA,
        pltpu.SemaphoreType.DMA,
        pltpu.SemaphoreType.DMA,
        pltpu.SemaphoreType.DMA,
        pltpu.SemaphoreType.DMA,
        pltpu.SemaphoreType.DMA,
        pltpu.SemaphoreType.DMA,
    ],
)
def _edge_kernel(row2_hbm, col_hbm, y_hbm, s_out,
                 sS, rowb0, rowb1, colb0, colb1, gb, zb,
                 isem0, isem1, g0, g1, g2, g3, s0, s1, s2, s3):
    core = lax.axis_index("c")
    s = lax.axis_index("s")

    _fill2d(zb, 128, HD, 0.0)
    _zero_stripe(sS, zb, s * STRIPE)
    plsc.subcore_barrier()

    rbase = core * ROWS_ALL + s * RPT
    cbase = s * RPT
    rowbs, colbs = (rowb0, rowb1), (colb0, colb1)
    isems = (isem0, isem1)
    gsems = (g0, g1, g2, g3)
    ssems = (s0, s1, s2, s3)

    def idx_copies(b, p):
        return (pltpu.make_async_copy(
                    row2_hbm.at[pl.ds(rbase + b * 4, 4)], rowbs[p], isems[p]),
                pltpu.make_async_copy(
                    col_hbm.at[pl.ds(cbase + b * 4, 4)], colbs[p], isems[p]))

    def gather(p, j):
        return pltpu.make_async_copy(
            y_hbm.at[rowbs[p].at[j]], gb.at[j], gsems[j])

    def scatter(p, j):
        return pltpu.make_async_copy(
            gb.at[j], sS.at[colbs[p].at[j]], ssems[j])

    def round_(b, p, first, last):
        for d in idx_copies(b, p):
            d.wait()
        if not last:
            for d in idx_copies(b + 1, 1 - p):
                d.start()
        for j in range(4):
            if not first:
                scatter(1 - p, j).wait()
            gather(p, j).start()
        for j in range(4):
            gather(p, j).wait()
            scatter(p, j).start(add=True)

    for d in idx_copies(0, 0):
        d.start()
    round_(0, 0, True, False)

    def pair(r, _):
        round_(2 * r + 1, 1, False, False)
        round_(2 * r + 2, 0, False, False)
        return None
    lax.fori_loop(0, (RB - 2) // 2, pair, None)
    round_(RB - 1, 1, False, True)
    for j in range(4):
        scatter(1, j).wait()

    plsc.subcore_barrier()
    lrow = s * STRIPE
    pltpu.sync_copy(sS.at[pl.ds(lrow, STRIPE)],
                    s_out.at[pl.ds(core * N_PAD + lrow, STRIPE)])


def _make_scale(scale):
    def body(s2_ref, dis_ref, acc_ref, y2_ref, accout_ref):
        d = dis_ref[:, 0:1]
        xa = s2_ref[0] * d
        xb = s2_ref[1] * d
        y2_ref[0] = xa * d
        y2_ref[1] = xb * d
        aa = acc_ref[:, :HD] + xa
        ab = acc_ref[:, HD:] + xb
        if scale != 1.0:
            aa = aa * scale
            ab = ab * scale
        accout_ref[...] = jnp.concatenate([aa, ab], axis=1)

    return pl.pallas_call(
        body,
        grid=(N_PAD // _K1_ROWS,),
        in_specs=[
            pl.BlockSpec((2, _K1_ROWS, HD), lambda i: (0, i, 0)),
            pl.BlockSpec((_K1_ROWS, 16), lambda i: (i, 0)),
            pl.BlockSpec((_K1_ROWS, D), lambda i: (i, 0)),
        ],
        out_specs=[
            pl.BlockSpec((2, _K1_ROWS, HD), lambda i: (0, i, 0)),
            pl.BlockSpec((_K1_ROWS, D), lambda i: (i, 0)),
        ],
        out_shape=[
            jax.ShapeDtypeStruct((2, N_PAD, HD), jnp.float32),
            jax.ShapeDtypeStruct((N_PAD, D), jnp.float32),
        ],
    )


_scale_mid = _make_scale(1.0)
_scale_last = _make_scale(1.0 / (NUM_LAYERS + 1) ** 2)


def kernel(edge_index, emb_users, emb_items):
    row = edge_index[0].astype(jnp.int32)
    col = edge_index[1].astype(jnp.int32)
    rowm = jnp.concatenate(
        [row, jnp.full((E_PAD - E,), PAD_ROW, jnp.int32)]).reshape(-1, 128)
    row2 = jnp.concatenate([rowm, rowm + N_PAD], axis=0)
    colp = jnp.concatenate(
        [col, jnp.full((E_PAD - E,), PAD_COL, jnp.int32)]).reshape(-1, 128)
    emb = jnp.concatenate([emb_users, emb_items], axis=0)
    emb_pad = jnp.concatenate(
        [emb, jnp.zeros((N_PAD - N, D), jnp.float32)], axis=0)

    deg2 = _deg_kernel(colp).reshape(2, N_PAD, 16)
    dis16, y2 = _dis_call(deg2, emb_pad)

    acc = emb_pad
    for l in range(NUM_LAYERS):
        s2 = _edge_kernel(row2, colp, y2.reshape(2 * N_PAD, HD))
        fn = _scale_last if l == NUM_LAYERS - 1 else _scale_mid
        y2, acc = fn(s2.reshape(2, N_PAD, HD), dis16, acc)

    return (acc[:NUM_USERS], emb_users, acc[NUM_USERS:N], emb_items)


# speedup vs baseline: 11.9640x; 1.2286x over previous
import functools

import jax
import jax.numpy as jnp
from jax import lax
from jax.experimental import pallas as pl
from jax.experimental.pallas import tpu as pltpu
from jax.experimental.pallas import tpu_sc as plsc

NUM_USERS = 25000
NUM_ITEMS = 25000
N = NUM_USERS + NUM_ITEMS
D = 64
HD = D // 2
E = 800000
NUM_LAYERS = 4

NC, NS, LANES = 2, 16, 16
N_PAD = 50176
STRIPE = N_PAD // NS

E_PAD = 819200
ROWS_ALL = E_PAD // 128
RPT = ROWS_ALL // NS
RB = RPT // 4
RPT0 = ROWS_ALL // (NC * NS)
RB0 = RPT0 // 4

PAD_ROW = N_PAD - 8
PAD_COL = N_PAD

_mesh = plsc.VectorSubcoreMesh(
    core_axis_name="c", subcore_axis_name="s", num_cores=NC, num_subcores=NS)
_sc_params = pltpu.CompilerParams(use_tc_tiling_on_sc=False)


def _vloop(n, body):
    lax.fori_loop(0, n, lambda i, _: (body(i), None)[1], None)


def _fill2d(ref, rows, cols, value):
    def step(i):
        r = i // (cols // LANES)
        q = i % (cols // LANES)
        ref[r, pl.ds(q * LANES, LANES)] = jnp.full((LANES,), value, jnp.float32)
    _vloop(rows * (cols // LANES), step)


def _zero_stripe(sS, zb, stripe_base):
    for k in range(STRIPE // 128):
        pltpu.sync_copy(zb, sS.at[pl.ds(stripe_base + k * 128, 128)])
    pltpu.sync_copy(zb.at[pl.ds(0, 64)],
                    sS.at[pl.ds(stripe_base + (STRIPE // 128) * 128, 64)])


@functools.partial(
    pl.kernel,
    out_type=jax.ShapeDtypeStruct((2 * N_PAD, 16), jnp.float32),
    mesh=_mesh,
    compiler_params=_sc_params,
    scratch_types=[
        pltpu.VMEM_SHARED((N_PAD + 8, 16), jnp.float32),
        pltpu.VMEM((4, 128), jnp.int32),
        pltpu.VMEM((4, 128), jnp.int32),
        pltpu.VMEM((128, 16), jnp.float32),
        pltpu.VMEM((128, 16), jnp.float32),
        pltpu.SemaphoreType.DMA,
        pltpu.SemaphoreType.DMA,
        pltpu.SemaphoreType.DMA,
    ],
)
def _deg_kernel(col_hbm, deg_out, dS, colb0, colb1, ones, zb,
                isem0, isem1, ssem):
    core = lax.axis_index("c")
    s = lax.axis_index("s")

    _fill2d(ones, 128, 16, 1.0)
    _fill2d(zb, 128, 16, 0.0)
    _zero_stripe(dS, zb, s * STRIPE)
    plsc.subcore_barrier()

    tbase = (core * NS + s) * RPT0
    colbs, isems = (colb0, colb1), (isem0, isem1)

    def idx_copy(b, p):
        return pltpu.make_async_copy(
            col_hbm.at[pl.ds(tbase + b * 4, 4)], colbs[p], isems[p])

    def round_(b, p, last):
        idx_copy(b, p).wait()
        if not last:
            idx_copy(b + 1, 1 - p).start()
        for j in range(4):
            pltpu.make_async_copy(
                ones, dS.at[colbs[p].at[j]], ssem).start(add=True)

    idx_copy(0, 0).start()
    round_(0, 0, False)

    def pair(r, _):
        round_(2 * r + 1, 1, False)
        round_(2 * r + 2, 0, False)
        return None
    lax.fori_loop(0, (RB0 - 2) // 2, pair, None)
    round_(RB0 - 1, 1, True)

    _vloop(RB0 * 4, lambda i: pltpu.make_async_copy(
        ones, dS.at[colb0.at[0]], ssem).wait())

    plsc.subcore_barrier()
    lrow = s * STRIPE
    pltpu.sync_copy(dS.at[pl.ds(lrow, STRIPE)],
                    deg_out.at[pl.ds(core * N_PAD + lrow, STRIPE)])


_K1_ROWS = 256


def _dis_body(deg_ref, emb_ref, dis2_ref, inv_ref, y2_ref):
    d = deg_ref[0, :, 0:1] + deg_ref[1, :, 0:1]
    disc = jnp.where(d > 0, lax.rsqrt(d), 0.0)
    dis2_ref[...] = jnp.broadcast_to(disc * disc, dis2_ref.shape)
    inv_ref[...] = jnp.broadcast_to(d * disc, inv_ref.shape)
    y2_ref[0] = emb_ref[:, :HD] * disc
    y2_ref[1] = emb_ref[:, HD:] * disc


_dis_call = pl.pallas_call(
    _dis_body,
    grid=(N_PAD // _K1_ROWS,),
    in_specs=[
        pl.BlockSpec((2, _K1_ROWS, 16), lambda i: (0, i, 0)),
        pl.BlockSpec((_K1_ROWS, D), lambda i: (i, 0)),
    ],
    out_specs=[
        pl.BlockSpec((_K1_ROWS, 16), lambda i: (i, 0)),
        pl.BlockSpec((_K1_ROWS, 16), lambda i: (i, 0)),
        pl.BlockSpec((2, _K1_ROWS, HD), lambda i: (0, i, 0)),
    ],
    out_shape=[
        jax.ShapeDtypeStruct((N_PAD, 16), jnp.float32),
        jax.ShapeDtypeStruct((N_PAD, 16), jnp.float32),
        jax.ShapeDtypeStruct((2, N_PAD, HD), jnp.float32),
    ],
)


@functools.partial(
    pl.kernel,
    out_type=jax.ShapeDtypeStruct((2 * N_PAD, HD), jnp.float32),
    mesh=_mesh,
    compiler_params=_sc_params,
    scratch_types=[
        pltpu.VMEM_SHARED((N_PAD + 8, HD), jnp.float32),
        pltpu.VMEM((4, 128), jnp.int32),
        pltpu.VMEM((4, 128), jnp.int32),
        pltpu.VMEM((4, 128), jnp.int32),
        pltpu.VMEM((4, 128), jnp.int32),
        pltpu.VMEM((4, 128, HD), jnp.float32),
        pltpu.VMEM((128, HD), jnp.float32),
        pltpu.VMEM((128, 16), jnp.float32),
        pltpu.VMEM((128, 16), jnp.float32),
        pltpu.SemaphoreType.DMA,
        pltpu.SemaphoreType.DMA,
        pltpu.SemaphoreType.DMA,
        pltpu.SemaphoreType.DMA,
        pltpu.SemaphoreType.DMA,
        pltpu.SemaphoreType.DMA,
        pltpu.SemaphoreType.DMA,
        pltpu.SemaphoreType.DMA,
        pltpu.SemaphoreType.DMA,
        pltpu.SemaphoreType.DMA,
    ],
)
def _edge_kernel(row2_hbm, col_hbm, y_hbm, dis2_hbm, y_out,
                 sS, rowb0, rowb1, colb0, colb1, gb, zb, d2b0, d2b1,
                 isem0, isem1, g0, g1, g2, g3, s0, s1, s2, s3):
    core = lax.axis_index("c")
    s = lax.axis_index("s")

    _fill2d(zb, 128, HD, 0.0)
    _zero_stripe(sS, zb, s * STRIPE)
    plsc.subcore_barrier()

    rbase = core * ROWS_ALL + s * RPT
    cbase = s * RPT
    rowbs, colbs = (rowb0, rowb1), (colb0, colb1)
    isems = (isem0, isem1)
    gsems = (g0, g1, g2, g3)
    ssems = (s0, s1, s2, s3)

    def idx_copies(b, p):
        return (pltpu.make_async_copy(
                    row2_hbm.at[pl.ds(rbase + b * 4, 4)], rowbs[p], isems[p]),
                pltpu.make_async_copy(
                    col_hbm.at[pl.ds(cbase + b * 4, 4)], colbs[p], isems[p]))

    def gather(p, j):
        return pltpu.make_async_copy(
            y_hbm.at[rowbs[p].at[j]], gb.at[j], gsems[j])

    def scatter(p, j):
        return pltpu.make_async_copy(
            gb.at[j], sS.at[colbs[p].at[j]], ssems[j])

    def round_(b, p, first, last):
        for d in idx_copies(b, p):
            d.wait()
        if not last:
            for d in idx_copies(b + 1, 1 - p):
                d.start()
        for j in range(4):
            if not first:
                scatter(1 - p, j).wait()
            gather(p, j).start()
        for j in range(4):
            gather(p, j).wait()
            scatter(p, j).start(add=True)

    for d in idx_copies(0, 0):
        d.start()
    round_(0, 0, True, False)

    def pair(r, _):
        round_(2 * r + 1, 1, False, False)
        round_(2 * r + 2, 0, False, False)
        return None
    lax.fori_loop(0, (RB - 2) // 2, pair, None)
    round_(RB - 1, 1, False, True)
    for j in range(4):
        scatter(1, j).wait()

    plsc.subcore_barrier()

    lrow = s * STRIPE
    obase = core * N_PAD + lrow
    nfull = STRIPE // 128
    tail = STRIPE - nfull * 128
    d2bs = (d2b0, d2b1)

    def d2copy(c, p, n):
        return pltpu.make_async_copy(
            dis2_hbm.at[pl.ds(lrow + c * 128, n)],
            d2bs[p].at[pl.ds(0, n)], isems[p])

    def outw(c, q, n):
        return pltpu.make_async_copy(
            gb.at[q].at[pl.ds(0, n)],
            y_out.at[pl.ds(obase + c * 128, n)], gsems[q])

    def mul_chunk(q, p, n):
        def step(e):
            dvec = d2bs[p][e, pl.ds(0, 16)]
            for h in range(2):
                gb[q, e, pl.ds(h * 16, 16)] = (
                    gb[q, e, pl.ds(h * 16, 16)] * dvec)
        _vloop(n, step)

    d2copy(0, 0, 128).start()
    for c in range(nfull):
        p = c % 2
        d2copy(c, p, 128).wait()
        if c + 1 < nfull:
            d2copy(c + 1, 1 - p, 128).start()
        else:
            d2copy(nfull, 1 - p, tail).start()
        if c >= 2:
            outw(c - 2, p, 128).wait()
        pltpu.sync_copy(sS.at[pl.ds(lrow + c * 128, 128)], gb.at[p])
        mul_chunk(p, p, 128)
        outw(c, p, 128).start()

    pt = nfull % 2
    d2copy(nfull, pt, tail).wait()
    outw(nfull - 2, pt, 128).wait()
    pltpu.sync_copy(sS.at[pl.ds(lrow + nfull * 128, tail)],
                    gb.at[pt].at[pl.ds(0, tail)])
    mul_chunk(pt, pt, tail)
    outw(nfull, pt, tail).start()
    outw(nfull - 1, 1 - pt, 128).wait()
    outw(nfull, pt, tail).wait()


_OUT_SCALE = 1.0 / (NUM_LAYERS + 1) ** 2


def _final_body(y1_ref, y2_ref, y3_ref, y4_ref, inv_ref, emb_ref, acc_ref):
    inv = inv_ref[:, 0:1]
    sa = (y1_ref[0] + y2_ref[0] + y3_ref[0] + y4_ref[0]) * inv
    sb = (y1_ref[1] + y2_ref[1] + y3_ref[1] + y4_ref[1]) * inv
    aa = (emb_ref[:, :HD] + sa) * _OUT_SCALE
    ab = (emb_ref[:, HD:] + sb) * _OUT_SCALE
    acc_ref[...] = jnp.concatenate([aa, ab], axis=1)


_final_call = pl.pallas_call(
    _final_body,
    grid=(N_PAD // _K1_ROWS,),
    in_specs=[pl.BlockSpec((2, _K1_ROWS, HD), lambda i: (0, i, 0))] * 4 + [
        pl.BlockSpec((_K1_ROWS, 16), lambda i: (i, 0)),
        pl.BlockSpec((_K1_ROWS, D), lambda i: (i, 0)),
    ],
    out_specs=pl.BlockSpec((_K1_ROWS, D), lambda i: (i, 0)),
    out_shape=jax.ShapeDtypeStruct((N_PAD, D), jnp.float32),
)


def kernel(edge_index, emb_users, emb_items):
    row = edge_index[0].astype(jnp.int32)
    col = edge_index[1].astype(jnp.int32)
    rowm = jnp.concatenate(
        [row, jnp.full((E_PAD - E,), PAD_ROW, jnp.int32)]).reshape(-1, 128)
    row2 = jnp.concatenate([rowm, rowm + N_PAD], axis=0)
    colp = jnp.concatenate(
        [col, jnp.full((E_PAD - E,), PAD_COL, jnp.int32)]).reshape(-1, 128)
    emb = jnp.concatenate([emb_users, emb_items], axis=0)
    emb_pad = jnp.concatenate(
        [emb, jnp.zeros((N_PAD - N, D), jnp.float32)], axis=0)

    deg2 = _deg_kernel(colp).reshape(2, N_PAD, 16)
    dis2t, inv16, y2 = _dis_call(deg2, emb_pad)

    y = y2.reshape(2 * N_PAD, HD)
    ys = []
    for _ in range(NUM_LAYERS):
        y = _edge_kernel(row2, colp, y, dis2t)
        ys.append(y.reshape(2, N_PAD, HD))

    acc = _final_call(ys[0], ys[1], ys[2], ys[3], inv16, emb_pad)
    return (acc[:NUM_USERS], emb_users, acc[NUM_USERS:N], emb_items)
